# Initial kernel scaffold; baseline (speedup 1.0000x reference)
import jax
import jax.numpy as jnp
from jax.experimental import pallas as pl


def kernel(edge_index, edge_energy, species, per_edge_scales):
    raise NotImplementedError("write your pallas kernel here")


# speedup vs baseline: 319.8434x; 319.8434x over previous
import functools
import math

import jax
import jax.numpy as jnp
from jax import lax
from jax.experimental import pallas as pl
from jax.experimental.pallas import tpu as pltpu
from jax.experimental.pallas import tpu_sc as plsc

_N_NODES = 100000
_N_EDGES = 6400000
_NUM_TYPES = 64
_FACTOR = 1.0 / math.sqrt(64.0)

_NC = 2
_NS = 16
_NW = _NC * _NS

_CHUNK_ROWS = 16
_CHUNK_COLS = 128
_CHUNK = _CHUNK_ROWS * _CHUNK_COLS
_N_CHUNKS = _N_EDGES // _CHUNK

_ACC_PAD = 100352
_ACC_SLICE = _ACC_PAD // _NS


def _sc_body(ctr_hbm, nbr_hbm, eng_hbm, spec_hbm, scal_hbm, out_hbm,
             ctr_v, nbr_v, eng_v, spec_v, scal_v, zero_v, acc):
    cid = lax.axis_index("c")
    sid = lax.axis_index("s")
    wid = sid * _NC + cid

    pltpu.sync_copy(spec_hbm, spec_v)
    pltpu.sync_copy(scal_hbm, scal_v)

    zeros16 = jnp.zeros((16,), jnp.float32)

    def _zero(i, carry):
        zero_v[pl.ds(i * 16, 16)] = zeros16
        return carry

    lax.fori_loop(0, _ACC_SLICE // 16, _zero, 0)
    pltpu.sync_copy(zero_v, acc.at[pl.ds(sid * _ACC_SLICE, _ACC_SLICE)])
    plsc.subcore_barrier()

    n_chunks = (_N_CHUNKS - wid + _NW - 1) // _NW

    def _chunk(i, carry):
        chunk = wid + i * _NW
        pltpu.sync_copy(ctr_hbm.at[chunk], ctr_v)
        pltpu.sync_copy(nbr_hbm.at[chunk], nbr_v)
        pltpu.sync_copy(eng_hbm.at[chunk], eng_v)

        def _row(j, carry2):
            def _col(c, carry3):
                sl = pl.ds(c * 16, 16)
                ci = ctr_v[j, sl]
                ni = nbr_v[j, sl]
                spc = plsc.load_gather(spec_v, [ci])
                spn = plsc.load_gather(spec_v, [ni])
                flat = spc * _NUM_TYPES + spn
                sc = plsc.load_gather(scal_v, [flat])
                eng_v[j, sl] = eng_v[j, sl] * sc
                return carry3

            return lax.fori_loop(0, _CHUNK_COLS // 16, _col, carry2)

        lax.fori_loop(0, _CHUNK_ROWS, _row, 0)

        def _srow(j, carry2):
            pltpu.sync_copy(eng_v.at[j], acc.at[ctr_v.at[j]], add=True)
            return carry2

        lax.fori_loop(0, _CHUNK_ROWS, _srow, 0)
        return carry

    lax.fori_loop(0, n_chunks, _chunk, 0)
    plsc.subcore_barrier()

    sl = pl.ds(sid * _ACC_SLICE, _ACC_SLICE)
    pltpu.sync_copy(acc.at[sl], out_hbm.at[cid, sl])


def _combine_body(p_ref, o_ref):
    o_ref[...] = (p_ref[0] + p_ref[1]) * _FACTOR


def kernel(edge_index, edge_energy, species, per_edge_scales):
    ctr = edge_index[0].reshape(_N_CHUNKS, _CHUNK_ROWS, _CHUNK_COLS)
    nbr = edge_index[1].reshape(_N_CHUNKS, _CHUNK_ROWS, _CHUNK_COLS)
    eng = edge_energy.reshape(_N_CHUNKS, _CHUNK_ROWS, _CHUNK_COLS)
    spec = species.reshape(_N_NODES)
    scal = per_edge_scales.reshape(_NUM_TYPES * _NUM_TYPES)

    mesh = plsc.VectorSubcoreMesh(
        core_axis_name="c", subcore_axis_name="s",
        num_cores=_NC, num_subcores=_NS)

    partials = pl.kernel(
        _sc_body,
        out_type=jax.ShapeDtypeStruct((_NC, _ACC_PAD), jnp.float32),
        mesh=mesh,
        compiler_params=pltpu.CompilerParams(needs_layout_passes=False),
        scratch_types=[
            pltpu.VMEM((_CHUNK_ROWS, _CHUNK_COLS), jnp.int32),
            pltpu.VMEM((_CHUNK_ROWS, _CHUNK_COLS), jnp.int32),
            pltpu.VMEM((_CHUNK_ROWS, _CHUNK_COLS), jnp.float32),
            pltpu.VMEM((_N_NODES,), jnp.int32),
            pltpu.VMEM((_NUM_TYPES * _NUM_TYPES,), jnp.float32),
            pltpu.VMEM((_ACC_SLICE,), jnp.float32),
            pltpu.VMEM_SHARED((_ACC_PAD,), jnp.float32),
        ],
    )(ctr, nbr, eng, spec, scal)

    combined = pl.pallas_call(
        _combine_body,
        out_shape=jax.ShapeDtypeStruct((_ACC_PAD // 128, 128), jnp.float32),
    )(partials.reshape(_NC, _ACC_PAD // 128, 128))

    return combined.reshape(_ACC_PAD)[:_N_NODES, None]


# speedup vs baseline: 506.1736x; 1.5826x over previous
import functools
import math

import jax
import jax.numpy as jnp
from jax import lax
from jax.experimental import pallas as pl
from jax.experimental.pallas import tpu as pltpu
from jax.experimental.pallas import tpu_sc as plsc

_N_NODES = 100000
_N_EDGES = 6400000
_NUM_TYPES = 64
_FACTOR = 1.0 / math.sqrt(64.0)

_NC = 2
_NS = 16
_NW = _NC * _NS

_CHUNK_ROWS = 16
_CHUNK_COLS = 128
_CHUNK = _CHUNK_ROWS * _CHUNK_COLS
_N_CHUNKS = _N_EDGES // _CHUNK

_ACC_PAD = 100352
_ACC_SLICE = _ACC_PAD // _NS


def _sc_body(ctr_hbm, nbr_hbm, eng_hbm, spec_hbm, scal_hbm, out_hbm,
             ctr_v, nbr_v, eng_v, spec_v, scal_v, zero_v, acc,
             ld_sem, st_sem):
    cid = lax.axis_index("c")
    sid = lax.axis_index("s")
    wid = sid * _NC + cid

    pltpu.sync_copy(spec_hbm, spec_v)
    pltpu.sync_copy(scal_hbm, scal_v)

    zeros16 = jnp.zeros((16,), jnp.float32)

    def _zero(i, carry):
        zero_v[pl.ds(i * 16, 16)] = zeros16
        return carry

    lax.fori_loop(0, _ACC_SLICE // 16, _zero, 0)
    pltpu.sync_copy(zero_v, acc.at[pl.ds(sid * _ACC_SLICE, _ACC_SLICE)])
    plsc.subcore_barrier()

    n_chunks = (_N_CHUNKS - wid + _NW - 1) // _NW

    def _issue_loads(chunk, p):
        pltpu.async_copy(ctr_hbm.at[chunk], ctr_v.at[p], ld_sem.at[p])
        pltpu.async_copy(nbr_hbm.at[chunk], nbr_v.at[p], ld_sem.at[p])
        pltpu.async_copy(eng_hbm.at[chunk], eng_v.at[p], ld_sem.at[p])

    def _wait_loads(chunk, p):
        pltpu.make_async_copy(ctr_hbm.at[chunk], ctr_v.at[p],
                              ld_sem.at[p]).wait()
        pltpu.make_async_copy(nbr_hbm.at[chunk], nbr_v.at[p],
                              ld_sem.at[p]).wait()
        pltpu.make_async_copy(eng_hbm.at[chunk], eng_v.at[p],
                              ld_sem.at[p]).wait()

    _issue_loads(wid, 0)

    def _chunk(i, carry):
        p = lax.rem(i, 2)
        chunk = wid + i * _NW

        @pl.when(i + 1 < n_chunks)
        def _prefetch():
            _issue_loads(chunk + _NW, 1 - p)

        _wait_loads(chunk, p)

        def _row(j, carry2):
            for c in range(_CHUNK_COLS // 16):
                sl = pl.ds(c * 16, 16)
                ci = ctr_v[p, j, sl]
                ni = nbr_v[p, j, sl]
                spc = plsc.load_gather(spec_v, [ci])
                spn = plsc.load_gather(spec_v, [ni])
                flat = spc * _NUM_TYPES + spn
                sc = plsc.load_gather(scal_v, [flat])
                eng_v[p, j, sl] = eng_v[p, j, sl] * sc
            return carry2

        lax.fori_loop(0, _CHUNK_ROWS, _row, 0)

        descs = []
        for j in range(_CHUNK_ROWS):
            descs.append(pltpu.async_copy(
                eng_v.at[p].at[j], acc.at[ctr_v.at[p].at[j]], st_sem,
                add=True))
        for d in descs:
            d.wait()
        return carry

    lax.fori_loop(0, n_chunks, _chunk, 0)
    plsc.subcore_barrier()

    sl = pl.ds(sid * _ACC_SLICE, _ACC_SLICE)
    pltpu.sync_copy(acc.at[sl], out_hbm.at[cid, sl])


def _combine_body(p_ref, o_ref):
    o_ref[...] = (p_ref[0] + p_ref[1]) * _FACTOR


def kernel(edge_index, edge_energy, species, per_edge_scales):
    ctr = edge_index[0].reshape(_N_CHUNKS, _CHUNK_ROWS, _CHUNK_COLS)
    nbr = edge_index[1].reshape(_N_CHUNKS, _CHUNK_ROWS, _CHUNK_COLS)
    eng = edge_energy.reshape(_N_CHUNKS, _CHUNK_ROWS, _CHUNK_COLS)
    spec = species.reshape(_N_NODES)
    scal = per_edge_scales.reshape(_NUM_TYPES * _NUM_TYPES)

    mesh = plsc.VectorSubcoreMesh(
        core_axis_name="c", subcore_axis_name="s",
        num_cores=_NC, num_subcores=_NS)

    partials = pl.kernel(
        _sc_body,
        out_type=jax.ShapeDtypeStruct((_NC, _ACC_PAD), jnp.float32),
        mesh=mesh,
        compiler_params=pltpu.CompilerParams(needs_layout_passes=False),
        scratch_types=[
            pltpu.VMEM((2, _CHUNK_ROWS, _CHUNK_COLS), jnp.int32),
            pltpu.VMEM((2, _CHUNK_ROWS, _CHUNK_COLS), jnp.int32),
            pltpu.VMEM((2, _CHUNK_ROWS, _CHUNK_COLS), jnp.float32),
            pltpu.VMEM((_N_NODES,), jnp.int32),
            pltpu.VMEM((_NUM_TYPES * _NUM_TYPES,), jnp.float32),
            pltpu.VMEM((_ACC_SLICE,), jnp.float32),
            pltpu.VMEM_SHARED((_ACC_PAD,), jnp.float32),
            pltpu.SemaphoreType.DMA((2,)),
            pltpu.SemaphoreType.DMA,
        ],
    )(ctr, nbr, eng, spec, scal)

    combined = pl.pallas_call(
        _combine_body,
        out_shape=jax.ShapeDtypeStruct((_ACC_PAD // 128, 128), jnp.float32),
    )(partials.reshape(_NC, _ACC_PAD // 128, 128))

    return combined.reshape(_ACC_PAD)[:_N_NODES, None]


# speedup vs baseline: 619.3029x; 1.2235x over previous
import functools
import math

import jax
import jax.numpy as jnp
from jax import lax
from jax.experimental import pallas as pl
from jax.experimental.pallas import tpu as pltpu
from jax.experimental.pallas import tpu_sc as plsc

_N_NODES = 100000
_N_EDGES = 6400000
_NUM_TYPES = 64
_FACTOR = 1.0 / math.sqrt(64.0)

_NC = 2
_NS = 16
_NW = _NC * _NS

_CHUNK_ROWS = 16
_CHUNK_COLS = 128
_CHUNK = _CHUNK_ROWS * _CHUNK_COLS
_N_CHUNKS = _N_EDGES // _CHUNK

_ACC_PAD = 100352
_ACC_SLICE = _ACC_PAD // _NS


def _sc_body(ei_hbm, eng_hbm, spec_hbm, scal_hbm, out_hbm,
             ctr_v, nbr_v, eng_v, spec_v, scal_v, zero_v, acc,
             ld_sem, st_sem):
    cid = lax.axis_index("c")
    sid = lax.axis_index("s")
    wid = sid * _NC + cid

    pltpu.sync_copy(spec_hbm, spec_v)
    pltpu.sync_copy(scal_hbm, scal_v)

    zeros16 = jnp.zeros((16,), jnp.float32)

    def _zero(i, carry):
        zero_v[pl.ds(i * 16, 16)] = zeros16
        return carry

    lax.fori_loop(0, _ACC_SLICE // 16, _zero, 0)
    pltpu.sync_copy(zero_v, acc.at[pl.ds(sid * _ACC_SLICE, _ACC_SLICE)])
    plsc.subcore_barrier()

    n_chunks = (_N_CHUNKS - wid + _NW - 1) // _NW

    def _issue_loads(chunk, p):
        pltpu.async_copy(ei_hbm.at[0].at[chunk], ctr_v.at[p], ld_sem.at[p])
        pltpu.async_copy(ei_hbm.at[1].at[chunk], nbr_v.at[p], ld_sem.at[p])
        pltpu.async_copy(eng_hbm.at[chunk], eng_v.at[p], ld_sem.at[p])

    def _wait_loads(chunk, p):
        pltpu.make_async_copy(ei_hbm.at[0].at[chunk], ctr_v.at[p],
                              ld_sem.at[p]).wait()
        pltpu.make_async_copy(ei_hbm.at[1].at[chunk], nbr_v.at[p],
                              ld_sem.at[p]).wait()
        pltpu.make_async_copy(eng_hbm.at[chunk], eng_v.at[p],
                              ld_sem.at[p]).wait()

    def _drain_scatters(p):
        for j in range(_CHUNK_ROWS):
            pltpu.make_async_copy(
                eng_v.at[p].at[j], acc.at[ctr_v.at[p].at[j]],
                st_sem.at[p]).wait()

    _issue_loads(wid, 0)

    def _chunk(i, carry):
        p = lax.rem(i, 2)
        chunk = wid + i * _NW

        @pl.when(i > 0)
        def _drain_prev():
            _drain_scatters(1 - p)

        @pl.when(i + 1 < n_chunks)
        def _prefetch():
            _issue_loads(chunk + _NW, 1 - p)

        _wait_loads(chunk, p)

        def _row(j, carry2):
            for c in range(_CHUNK_COLS // 16):
                sl = pl.ds(c * 16, 16)
                ci = ctr_v[p, j, sl]
                ni = nbr_v[p, j, sl]
                spc = plsc.load_gather(spec_v, [ci])
                spn = plsc.load_gather(spec_v, [ni])
                flat = spc * _NUM_TYPES + spn
                sc = plsc.load_gather(scal_v, [flat])
                eng_v[p, j, sl] = eng_v[p, j, sl] * sc
            pltpu.async_copy(
                eng_v.at[p].at[j], acc.at[ctr_v.at[p].at[j]], st_sem.at[p],
                add=True)
            return carry2

        lax.fori_loop(0, _CHUNK_ROWS, _row, 0)
        return carry

    lax.fori_loop(0, n_chunks, _chunk, 0)
    _drain_scatters(lax.rem(n_chunks - 1, 2))
    plsc.subcore_barrier()

    sl = pl.ds(sid * _ACC_SLICE, _ACC_SLICE)
    pltpu.sync_copy(acc.at[sl], out_hbm.at[cid, sl])


def _combine_body(p_ref, o_ref):
    o_ref[...] = (p_ref[0] + p_ref[1]) * _FACTOR


def kernel(edge_index, edge_energy, species, per_edge_scales):
    ei = edge_index.reshape(2, _N_CHUNKS, _CHUNK_ROWS, _CHUNK_COLS)
    eng = edge_energy.reshape(_N_CHUNKS, _CHUNK_ROWS, _CHUNK_COLS)
    spec = species.reshape(_N_NODES)
    scal = per_edge_scales.reshape(_NUM_TYPES * _NUM_TYPES)

    mesh = plsc.VectorSubcoreMesh(
        core_axis_name="c", subcore_axis_name="s",
        num_cores=_NC, num_subcores=_NS)

    partials = pl.kernel(
        _sc_body,
        out_type=jax.ShapeDtypeStruct((_NC, _ACC_PAD), jnp.float32),
        mesh=mesh,
        compiler_params=pltpu.CompilerParams(needs_layout_passes=False),
        scratch_types=[
            pltpu.VMEM((2, _CHUNK_ROWS, _CHUNK_COLS), jnp.int32),
            pltpu.VMEM((2, _CHUNK_ROWS, _CHUNK_COLS), jnp.int32),
            pltpu.VMEM((2, _CHUNK_ROWS, _CHUNK_COLS), jnp.float32),
            pltpu.VMEM((_N_NODES,), jnp.int32),
            pltpu.VMEM((_NUM_TYPES * _NUM_TYPES,), jnp.float32),
            pltpu.VMEM((_ACC_SLICE,), jnp.float32),
            pltpu.VMEM_SHARED((_ACC_PAD,), jnp.float32),
            pltpu.SemaphoreType.DMA((2,)),
            pltpu.SemaphoreType.DMA((2,)),
        ],
    )(ei, eng, spec, scal)

    combined = pl.pallas_call(
        _combine_body,
        out_shape=jax.ShapeDtypeStruct((_ACC_PAD // 128, 128), jnp.float32),
    )(partials.reshape(_NC, _ACC_PAD // 128, 128))

    return combined.reshape(_ACC_PAD)[:_N_NODES, None]


# speedup vs baseline: 686.6208x; 1.1087x over previous
import functools
import math

import jax
import jax.numpy as jnp
from jax import lax
from jax.experimental import pallas as pl
from jax.experimental.pallas import tpu as pltpu
from jax.experimental.pallas import tpu_sc as plsc

_N_NODES = 100000
_N_EDGES = 6400000
_NUM_TYPES = 64
_FACTOR = 1.0 / math.sqrt(64.0)

_NC = 2
_NS = 16
_NW = _NC * _NS

_CHUNK = 2048
_ROW = 128
_N_CHUNKS = _N_EDGES // _CHUNK

_ACC_PAD = 100352
_ACC_SLICE = _ACC_PAD // _NS


def _sc_body(ei_hbm, eng_hbm, spec_hbm, scal_hbm, out_hbm,
             ctr_v, nbr_v, eng_v, spec_v, scal_v, zero_v, acc,
             ld_sem, st_sem):
    cid = lax.axis_index("c")
    sid = lax.axis_index("s")
    wid = sid * _NC + cid

    pltpu.sync_copy(spec_hbm, spec_v)
    pltpu.sync_copy(scal_hbm, scal_v)

    zeros16 = jnp.zeros((16,), jnp.float32)

    def _zero(i, carry):
        zero_v[pl.ds(i * 16, 16)] = zeros16
        return carry

    lax.fori_loop(0, _ACC_SLICE // 16, _zero, 0)
    pltpu.sync_copy(zero_v, acc.at[pl.ds(sid * _ACC_SLICE, _ACC_SLICE)])
    plsc.subcore_barrier()

    n_chunks = (_N_CHUNKS - wid + _NW - 1) // _NW

    def _issue_loads(chunk, p):
        sl = pl.ds(chunk * _CHUNK, _CHUNK)
        pltpu.async_copy(ei_hbm.at[0].at[sl], ctr_v.at[p], ld_sem.at[p])
        pltpu.async_copy(ei_hbm.at[1].at[sl], nbr_v.at[p], ld_sem.at[p])
        pltpu.async_copy(eng_hbm.at[sl], eng_v.at[p], ld_sem.at[p])

    def _wait_loads(chunk, p):
        sl = pl.ds(chunk * _CHUNK, _CHUNK)
        pltpu.make_async_copy(ei_hbm.at[0].at[sl], ctr_v.at[p],
                              ld_sem.at[p]).wait()
        pltpu.make_async_copy(ei_hbm.at[1].at[sl], nbr_v.at[p],
                              ld_sem.at[p]).wait()
        pltpu.make_async_copy(eng_hbm.at[sl], eng_v.at[p],
                              ld_sem.at[p]).wait()

    def _drain_scatters(p):
        for j in range(_CHUNK // _ROW):
            rsl = pl.ds(j * _ROW, _ROW)
            pltpu.make_async_copy(
                eng_v.at[p].at[rsl], acc.at[ctr_v.at[p].at[rsl]],
                st_sem.at[p]).wait()

    _issue_loads(wid, 0)

    def _chunk(i, carry):
        p = lax.rem(i, 2)
        chunk = wid + i * _NW

        @pl.when(i > 0)
        def _drain_prev():
            _drain_scatters(1 - p)

        @pl.when(i + 1 < n_chunks)
        def _prefetch():
            _issue_loads(chunk + _NW, 1 - p)

        _wait_loads(chunk, p)

        def _row(j, carry2):
            base = j * _ROW
            for c in range(_ROW // 16):
                sl = pl.ds(base + c * 16, 16)
                ci = ctr_v[p, sl]
                ni = nbr_v[p, sl]
                spc = plsc.load_gather(spec_v, [ci])
                spn = plsc.load_gather(spec_v, [ni])
                flat = spc * _NUM_TYPES + spn
                sc = plsc.load_gather(scal_v, [flat])
                eng_v[p, sl] = eng_v[p, sl] * sc
            rsl = pl.ds(pl.multiple_of(base, _ROW), _ROW)
            pltpu.async_copy(
                eng_v.at[p].at[rsl], acc.at[ctr_v.at[p].at[rsl]],
                st_sem.at[p], add=True)
            return carry2

        lax.fori_loop(0, _CHUNK // _ROW, _row, 0)
        return carry

    lax.fori_loop(0, n_chunks, _chunk, 0)
    _drain_scatters(lax.rem(n_chunks - 1, 2))
    plsc.subcore_barrier()

    sl = pl.ds(sid * _ACC_SLICE, _ACC_SLICE)
    pltpu.sync_copy(acc.at[sl], out_hbm.at[cid, sl])


def _combine_body(p_ref, o_ref):
    o_ref[...] = (p_ref[0] + p_ref[1]) * _FACTOR


def kernel(edge_index, edge_energy, species, per_edge_scales):
    eng = edge_energy.reshape(_N_EDGES)
    spec = species.reshape(_N_NODES)
    scal = per_edge_scales.reshape(_NUM_TYPES * _NUM_TYPES)

    mesh = plsc.VectorSubcoreMesh(
        core_axis_name="c", subcore_axis_name="s",
        num_cores=_NC, num_subcores=_NS)

    partials = pl.kernel(
        _sc_body,
        out_type=jax.ShapeDtypeStruct((_NC, _ACC_PAD), jnp.float32),
        mesh=mesh,
        compiler_params=pltpu.CompilerParams(needs_layout_passes=False),
        scratch_types=[
            pltpu.VMEM((2, _CHUNK), jnp.int32),
            pltpu.VMEM((2, _CHUNK), jnp.int32),
            pltpu.VMEM((2, _CHUNK), jnp.float32),
            pltpu.VMEM((_N_NODES,), jnp.int32),
            pltpu.VMEM((_NUM_TYPES * _NUM_TYPES,), jnp.float32),
            pltpu.VMEM((_ACC_SLICE,), jnp.float32),
            pltpu.VMEM_SHARED((_ACC_PAD,), jnp.float32),
            pltpu.SemaphoreType.DMA((2,)),
            pltpu.SemaphoreType.DMA((2,)),
        ],
    )(edge_index, eng, spec, scal)

    combined = pl.pallas_call(
        _combine_body,
        out_shape=jax.ShapeDtypeStruct((_ACC_PAD // 128, 128), jnp.float32),
    )(partials.reshape(_NC, _ACC_PAD // 128, 128))

    return combined.reshape(_ACC_PAD)[:_N_NODES, None]


# speedup vs baseline: 1610.8985x; 2.3461x over previous
import functools
import math

import jax
import jax.numpy as jnp
from jax import lax
from jax.experimental import pallas as pl
from jax.experimental.pallas import tpu as pltpu
from jax.experimental.pallas import tpu_sc as plsc

_N_NODES = 100000
_N_EDGES = 6400000
_NUM_TYPES = 64
_FACTOR = 1.0 / math.sqrt(64.0)

_NC = 2
_NS = 16
_NW = _NC * _NS

_CHUNK = 2048
_ROW = 128
_N_CHUNKS = _N_EDGES // _CHUNK

_ACC_PAD = 100352
_ACC_SLICE = _ACC_PAD // _NS


def _sc_body(ei_hbm, eng_hbm, spec_hbm, scal_hbm, out_hbm,
             ctr_v, nbr_v, eng_v, spec_v, scal_v, zero_v, acc,
             ld_sem, st_sem):
    cid = lax.axis_index("c")
    sid = lax.axis_index("s")
    wid = sid * _NC + cid

    pltpu.sync_copy(spec_hbm, spec_v)
    pltpu.sync_copy(scal_hbm, scal_v)

    zeros16 = jnp.zeros((16,), jnp.float32)

    def _zero(i, carry):
        zero_v[pl.ds(i * 16, 16)] = zeros16
        return carry

    lax.fori_loop(0, _ACC_SLICE // 16, _zero, 0)
    pltpu.sync_copy(zero_v, acc.at[pl.ds(sid * _ACC_SLICE, _ACC_SLICE)])
    plsc.subcore_barrier()

    n_chunks = (_N_CHUNKS - wid + _NW - 1) // _NW

    def _issue_loads(chunk, p):
        sl = pl.ds(chunk * _CHUNK, _CHUNK)
        pltpu.async_copy(ei_hbm.at[0].at[sl], ctr_v.at[p], ld_sem.at[p])
        pltpu.async_copy(ei_hbm.at[1].at[sl], nbr_v.at[p], ld_sem.at[p])
        pltpu.async_copy(eng_hbm.at[sl], eng_v.at[p], ld_sem.at[p])

    def _wait_loads(chunk, p):
        sl = pl.ds(chunk * _CHUNK, _CHUNK)
        pltpu.make_async_copy(ei_hbm.at[0].at[sl], ctr_v.at[p],
                              ld_sem.at[p]).wait()
        pltpu.make_async_copy(ei_hbm.at[1].at[sl], nbr_v.at[p],
                              ld_sem.at[p]).wait()
        pltpu.make_async_copy(eng_hbm.at[sl], eng_v.at[p],
                              ld_sem.at[p]).wait()

    def _drain_scatters(p):
        for j in range(_CHUNK // _ROW):
            rsl = pl.ds(j * _ROW, _ROW)
            pltpu.make_async_copy(
                eng_v.at[p].at[rsl], acc.at[ctr_v.at[p].at[rsl]],
                st_sem.at[p]).wait()

    _issue_loads(wid, 0)

    def _chunk(i, carry):
        p = lax.rem(i, 2)
        chunk = wid + i * _NW

        @pl.when(i > 0)
        def _drain_prev():
            _drain_scatters(1 - p)

        @pl.when(i + 1 < n_chunks)
        def _prefetch():
            _issue_loads(chunk + _NW, 1 - p)

        _wait_loads(chunk, p)

        def _row(j, carry2):
            base = j * _ROW
            ngrp = _ROW // 16
            sls = [pl.ds(base + c * 16, 16) for c in range(ngrp)]
            cis = [ctr_v[p, sl] for sl in sls]
            nis = [nbr_v[p, sl] for sl in sls]
            spcs = [plsc.load_gather(spec_v, [ci]) for ci in cis]
            spns = [plsc.load_gather(spec_v, [ni]) for ni in nis]
            flats = [spc * _NUM_TYPES + spn
                     for spc, spn in zip(spcs, spns)]
            scs = [plsc.load_gather(scal_v, [flat]) for flat in flats]
            engs = [eng_v[p, sl] for sl in sls]
            for sl, e, sc in zip(sls, engs, scs):
                eng_v[p, sl] = e * sc
            rsl = pl.ds(pl.multiple_of(base, _ROW), _ROW)
            pltpu.async_copy(
                eng_v.at[p].at[rsl], acc.at[ctr_v.at[p].at[rsl]],
                st_sem.at[p], add=True)
            return carry2

        lax.fori_loop(0, _CHUNK // _ROW, _row, 0)
        return carry

    lax.fori_loop(0, n_chunks, _chunk, 0)
    _drain_scatters(lax.rem(n_chunks - 1, 2))
    plsc.subcore_barrier()

    sl = pl.ds(sid * _ACC_SLICE, _ACC_SLICE)
    pltpu.sync_copy(acc.at[sl], out_hbm.at[cid, sl])


def _combine_body(p_ref, o_ref):
    o_ref[...] = (p_ref[0] + p_ref[1]) * _FACTOR


def kernel(edge_index, edge_energy, species, per_edge_scales):
    eng = edge_energy.reshape(_N_EDGES)
    spec = species.reshape(_N_NODES)
    scal = per_edge_scales.reshape(_NUM_TYPES * _NUM_TYPES)

    mesh = plsc.VectorSubcoreMesh(
        core_axis_name="c", subcore_axis_name="s",
        num_cores=_NC, num_subcores=_NS)

    partials = pl.kernel(
        _sc_body,
        out_type=jax.ShapeDtypeStruct((_NC, _ACC_PAD), jnp.float32),
        mesh=mesh,
        compiler_params=pltpu.CompilerParams(needs_layout_passes=False),
        scratch_types=[
            pltpu.VMEM((2, _CHUNK), jnp.int32),
            pltpu.VMEM((2, _CHUNK), jnp.int32),
            pltpu.VMEM((2, _CHUNK), jnp.float32),
            pltpu.VMEM((_N_NODES,), jnp.int32),
            pltpu.VMEM((_NUM_TYPES * _NUM_TYPES,), jnp.float32),
            pltpu.VMEM((_ACC_SLICE,), jnp.float32),
            pltpu.VMEM_SHARED((_ACC_PAD,), jnp.float32),
            pltpu.SemaphoreType.DMA((2,)),
            pltpu.SemaphoreType.DMA((2,)),
        ],
    )(edge_index, eng, spec, scal)

    combined = pl.pallas_call(
        _combine_body,
        out_shape=jax.ShapeDtypeStruct((_ACC_PAD // 128, 128), jnp.float32),
    )(partials.reshape(_NC, _ACC_PAD // 128, 128))

    return combined.reshape(_ACC_PAD)[:_N_NODES, None]


# speedup vs baseline: 1646.2059x; 1.0219x over previous
import functools
import math

import jax
import jax.numpy as jnp
from jax import lax
from jax.experimental import pallas as pl
from jax.experimental.pallas import tpu as pltpu
from jax.experimental.pallas import tpu_sc as plsc

_N_NODES = 100000
_N_EDGES = 6400000
_NUM_TYPES = 64
_FACTOR = 1.0 / math.sqrt(64.0)

_NC = 2
_NS = 16
_NW = _NC * _NS

_CHUNK = 2048
_ROW = 128
_N_CHUNKS = _N_EDGES // _CHUNK

_ACC_PAD = 100352
_ACC_SLICE = _ACC_PAD // _NS


def _sc_body(ei_hbm, eng_hbm, spec_hbm, scal_hbm, out_hbm,
             ctr_v, nbr_v, eng_v, spec_v, scal_v, zero_v, acc,
             ld_sem, st_sem, tb_sem):
    cid = lax.axis_index("c")
    sid = lax.axis_index("s")
    wid = sid * _NC + cid

    n_chunks = (_N_CHUNKS - wid + _NW - 1) // _NW

    def _issue_loads(chunk, p):
        sl = pl.ds(chunk * _CHUNK, _CHUNK)
        pltpu.async_copy(ei_hbm.at[0].at[sl], ctr_v.at[p], ld_sem.at[p])
        pltpu.async_copy(ei_hbm.at[1].at[sl], nbr_v.at[p], ld_sem.at[p])
        pltpu.async_copy(eng_hbm.at[sl], eng_v.at[p], ld_sem.at[p])

    def _wait_loads(chunk, p):
        sl = pl.ds(chunk * _CHUNK, _CHUNK)
        pltpu.make_async_copy(ei_hbm.at[0].at[sl], ctr_v.at[p],
                              ld_sem.at[p]).wait()
        pltpu.make_async_copy(ei_hbm.at[1].at[sl], nbr_v.at[p],
                              ld_sem.at[p]).wait()
        pltpu.make_async_copy(eng_hbm.at[sl], eng_v.at[p],
                              ld_sem.at[p]).wait()

    def _drain_scatters(p):
        for j in range(_CHUNK // _ROW):
            rsl = pl.ds(j * _ROW, _ROW)
            pltpu.make_async_copy(
                eng_v.at[p].at[rsl], acc.at[ctr_v.at[p].at[rsl]],
                st_sem.at[p]).wait()

    _issue_loads(wid, 0)
    spec_copy = pltpu.async_copy(spec_hbm, spec_v, tb_sem)
    scal_copy = pltpu.async_copy(scal_hbm, scal_v, tb_sem)

    zeros16 = jnp.zeros((16,), jnp.float32)

    def _zero(i, carry):
        zero_v[pl.ds(i * 16, 16)] = zeros16
        return carry

    lax.fori_loop(0, _ACC_SLICE // 16, _zero, 0)
    pltpu.sync_copy(zero_v, acc.at[pl.ds(sid * _ACC_SLICE, _ACC_SLICE)])
    spec_copy.wait()
    scal_copy.wait()
    plsc.subcore_barrier()

    def _chunk(i, carry):
        p = lax.rem(i, 2)
        chunk = wid + i * _NW

        @pl.when(i > 0)
        def _drain_prev():
            _drain_scatters(1 - p)

        @pl.when(i + 1 < n_chunks)
        def _prefetch():
            _issue_loads(chunk + _NW, 1 - p)

        _wait_loads(chunk, p)

        for j in range(_CHUNK // _ROW):
            base = j * _ROW
            ngrp = _ROW // 16
            sls = [pl.ds(base + c * 16, 16) for c in range(ngrp)]
            cis = [ctr_v[p, sl] for sl in sls]
            nis = [nbr_v[p, sl] for sl in sls]
            spcs = [plsc.load_gather(spec_v, [ci]) for ci in cis]
            spns = [plsc.load_gather(spec_v, [ni]) for ni in nis]
            flats = [spc * _NUM_TYPES + spn
                     for spc, spn in zip(spcs, spns)]
            scs = [plsc.load_gather(scal_v, [flat]) for flat in flats]
            engs = [eng_v[p, sl] for sl in sls]
            for sl, e, sc in zip(sls, engs, scs):
                eng_v[p, sl] = e * sc
            rsl = pl.ds(base, _ROW)
            pltpu.async_copy(
                eng_v.at[p].at[rsl], acc.at[ctr_v.at[p].at[rsl]],
                st_sem.at[p], add=True)
        return carry

    lax.fori_loop(0, n_chunks, _chunk, 0)
    _drain_scatters(lax.rem(n_chunks - 1, 2))
    plsc.subcore_barrier()

    sl = pl.ds(sid * _ACC_SLICE, _ACC_SLICE)
    pltpu.sync_copy(acc.at[sl], out_hbm.at[cid, sl])


def _combine_body(p_ref, o_ref):
    o_ref[...] = (p_ref[0] + p_ref[1]) * _FACTOR


def kernel(edge_index, edge_energy, species, per_edge_scales):
    eng = edge_energy.reshape(_N_EDGES)
    spec = species.reshape(_N_NODES)
    scal = per_edge_scales.reshape(_NUM_TYPES * _NUM_TYPES)

    mesh = plsc.VectorSubcoreMesh(
        core_axis_name="c", subcore_axis_name="s",
        num_cores=_NC, num_subcores=_NS)

    partials = pl.kernel(
        _sc_body,
        out_type=jax.ShapeDtypeStruct((_NC, _ACC_PAD), jnp.float32),
        mesh=mesh,
        compiler_params=pltpu.CompilerParams(needs_layout_passes=False),
        scratch_types=[
            pltpu.VMEM((2, _CHUNK), jnp.int32),
            pltpu.VMEM((2, _CHUNK), jnp.int32),
            pltpu.VMEM((2, _CHUNK), jnp.float32),
            pltpu.VMEM((_N_NODES,), jnp.int32),
            pltpu.VMEM((_NUM_TYPES * _NUM_TYPES,), jnp.float32),
            pltpu.VMEM((_ACC_SLICE,), jnp.float32),
            pltpu.VMEM_SHARED((_ACC_PAD,), jnp.float32),
            pltpu.SemaphoreType.DMA((2,)),
            pltpu.SemaphoreType.DMA((2,)),
            pltpu.SemaphoreType.DMA,
        ],
    )(edge_index, eng, spec, scal)

    combined = pl.pallas_call(
        _combine_body,
        out_shape=jax.ShapeDtypeStruct((_ACC_PAD // 128, 128), jnp.float32),
    )(partials.reshape(_NC, _ACC_PAD // 128, 128))

    return combined.reshape(_ACC_PAD)[:_N_NODES, None]


# speedup vs baseline: 1651.7082x; 1.0033x over previous
import functools
import math

import jax
import jax.numpy as jnp
from jax import lax
from jax.experimental import pallas as pl
from jax.experimental.pallas import tpu as pltpu
from jax.experimental.pallas import tpu_sc as plsc

_N_NODES = 100000
_N_EDGES = 6400000
_NUM_TYPES = 64
_FACTOR = 1.0 / math.sqrt(64.0)

_NC = 2
_NS = 16
_NW = _NC * _NS

_CHUNK = 2048
_ROW = 128
_N_CHUNKS = _N_EDGES // _CHUNK

_ACC_PAD = 100352
_ACC_SLICE = _ACC_PAD // _NS


def _sc_body(ei_hbm, eng_hbm, spec_hbm, scal_hbm, out_hbm,
             cnb_v, eng_v, spec_v, scal_v, zero_v, acc,
             ld_sem, st_sem, tb_sem):
    cid = lax.axis_index("c")
    sid = lax.axis_index("s")
    wid = sid * _NC + cid

    n_chunks = (_N_CHUNKS - wid + _NW - 1) // _NW

    def _issue_loads(chunk, p):
        sl = pl.ds(chunk * _CHUNK, _CHUNK)
        pltpu.async_copy(ei_hbm.at[:, sl], cnb_v.at[p], ld_sem.at[p])
        pltpu.async_copy(eng_hbm.at[sl], eng_v.at[p], ld_sem.at[p])

    def _wait_loads(chunk, p):
        sl = pl.ds(chunk * _CHUNK, _CHUNK)
        pltpu.make_async_copy(ei_hbm.at[:, sl], cnb_v.at[p],
                              ld_sem.at[p]).wait()
        pltpu.make_async_copy(eng_hbm.at[sl], eng_v.at[p],
                              ld_sem.at[p]).wait()

    def _drain_scatters(p):
        for j in range(_CHUNK // _ROW):
            rsl = pl.ds(j * _ROW, _ROW)
            pltpu.make_async_copy(
                eng_v.at[p].at[rsl], acc.at[cnb_v.at[p].at[0].at[rsl]],
                st_sem.at[p]).wait()

    _issue_loads(wid, 0)
    spec_copy = pltpu.async_copy(spec_hbm, spec_v, tb_sem)
    scal_copy = pltpu.async_copy(scal_hbm, scal_v, tb_sem)

    zeros16 = jnp.zeros((16,), jnp.float32)

    def _zero(i, carry):
        zero_v[pl.ds(i * 16, 16)] = zeros16
        return carry

    lax.fori_loop(0, _ACC_SLICE // 16, _zero, 0)
    pltpu.sync_copy(zero_v, acc.at[pl.ds(sid * _ACC_SLICE, _ACC_SLICE)])
    spec_copy.wait()
    scal_copy.wait()
    plsc.subcore_barrier()

    def _chunk(i, carry):
        p = lax.rem(i, 2)
        chunk = wid + i * _NW

        @pl.when(i > 0)
        def _drain_prev():
            _drain_scatters(1 - p)

        @pl.when(i + 1 < n_chunks)
        def _prefetch():
            _issue_loads(chunk + _NW, 1 - p)

        _wait_loads(chunk, p)

        for j in range(_CHUNK // _ROW):
            base = j * _ROW
            ngrp = _ROW // 16
            sls = [pl.ds(base + c * 16, 16) for c in range(ngrp)]
            cis = [cnb_v[p, 0, sl] for sl in sls]
            nis = [cnb_v[p, 1, sl] for sl in sls]
            spcs = [plsc.load_gather(spec_v, [ci]) for ci in cis]
            spns = [plsc.load_gather(spec_v, [ni]) for ni in nis]
            flats = [spc * _NUM_TYPES + spn
                     for spc, spn in zip(spcs, spns)]
            scs = [plsc.load_gather(scal_v, [flat]) for flat in flats]
            engs = [eng_v[p, sl] for sl in sls]
            for sl, e, sc in zip(sls, engs, scs):
                eng_v[p, sl] = e * sc
            rsl = pl.ds(base, _ROW)
            pltpu.async_copy(
                eng_v.at[p].at[rsl], acc.at[cnb_v.at[p].at[0].at[rsl]],
                st_sem.at[p], add=True)
        return carry

    lax.fori_loop(0, n_chunks, _chunk, 0)
    _drain_scatters(lax.rem(n_chunks - 1, 2))
    plsc.subcore_barrier()

    sl = pl.ds(sid * _ACC_SLICE, _ACC_SLICE)
    pltpu.sync_copy(acc.at[sl], out_hbm.at[cid, sl])


def _combine_body(p_ref, o_ref):
    o_ref[...] = (p_ref[0] + p_ref[1]) * _FACTOR


def kernel(edge_index, edge_energy, species, per_edge_scales):
    eng = edge_energy.reshape(_N_EDGES)
    spec = species.reshape(_N_NODES)
    scal = per_edge_scales.reshape(_NUM_TYPES * _NUM_TYPES)

    mesh = plsc.VectorSubcoreMesh(
        core_axis_name="c", subcore_axis_name="s",
        num_cores=_NC, num_subcores=_NS)

    partials = pl.kernel(
        _sc_body,
        out_type=jax.ShapeDtypeStruct((_NC, _ACC_PAD), jnp.float32),
        mesh=mesh,
        compiler_params=pltpu.CompilerParams(needs_layout_passes=False),
        scratch_types=[
            pltpu.VMEM((2, 2, _CHUNK), jnp.int32),
            pltpu.VMEM((2, _CHUNK), jnp.float32),
            pltpu.VMEM((_N_NODES,), jnp.int32),
            pltpu.VMEM((_NUM_TYPES * _NUM_TYPES,), jnp.float32),
            pltpu.VMEM((_ACC_SLICE,), jnp.float32),
            pltpu.VMEM_SHARED((_ACC_PAD,), jnp.float32),
            pltpu.SemaphoreType.DMA((2,)),
            pltpu.SemaphoreType.DMA((2,)),
            pltpu.SemaphoreType.DMA,
        ],
    )(edge_index, eng, spec, scal)

    combined = pl.pallas_call(
        _combine_body,
        out_shape=jax.ShapeDtypeStruct((_ACC_PAD // 128, 128), jnp.float32),
    )(partials.reshape(_NC, _ACC_PAD // 128, 128))

    return combined.reshape(_ACC_PAD)[:_N_NODES, None]
